# baseline (speedup 1.0000x reference)
import functools

import jax
import jax.numpy as jnp
from jax import lax
from jax.experimental import pallas as pl
from jax.experimental.pallas import tpu as pltpu
from jax.experimental.pallas import tpu_sc as plsc

N_CORES = 2
N_SUBCORES = 16
LANES = 16
NW = N_CORES * N_SUBCORES

B = 16384
BPW = B // NW
GROUPS = BPW // LANES

_mesh = plsc.VectorSubcoreMesh(core_axis_name="c", subcore_axis_name="s")


@functools.partial(
    pl.kernel,
    mesh=_mesh,
    out_type=jax.ShapeDtypeStruct((B * 16,), jnp.float32),
    compiler_params=pltpu.CompilerParams(needs_layout_passes=False),
    scratch_types=[
        pltpu.VMEM((BPW,), jnp.int32),
        pltpu.VMEM((BPW * 4,), jnp.int32),
        pltpu.VMEM((BPW * 3,), jnp.int32),
        pltpu.VMEM((BPW * 4,), jnp.float32),
        pltpu.VMEM((BPW * 3,), jnp.float32),
        pltpu.VMEM((BPW * 16,), jnp.float32),
        pltpu.SemaphoreType.DMA,
        pltpu.SemaphoreType.DMA,
    ],
)
def _pose_table_sc(idx_hbm, q_hbm, t_hbm, out_hbm,
                   idx_v, qix_v, tix_v, q_f, t_f, out_f, qsem, tsem):
    wid = lax.axis_index("s") * N_CORES + lax.axis_index("c")
    base = wid * BPW

    pltpu.sync_copy(idx_hbm.at[pl.ds(base, BPW)], idx_v)

    def build(g, carry):
        o = g * LANES
        i = idx_v[pl.ds(o, LANES)]
        i4 = i * 4
        qix_v[pl.ds(o, LANES)] = i4
        qix_v[pl.ds(BPW + o, LANES)] = i4 + 1
        qix_v[pl.ds(2 * BPW + o, LANES)] = i4 + 2
        qix_v[pl.ds(3 * BPW + o, LANES)] = i4 + 3
        i3 = i * 3
        tix_v[pl.ds(o, LANES)] = i3
        tix_v[pl.ds(BPW + o, LANES)] = i3 + 1
        tix_v[pl.ds(2 * BPW + o, LANES)] = i3 + 2
        return carry

    lax.fori_loop(0, 1, build, 0)

    qcp = pltpu.async_copy(q_hbm.at[pl.ds(0, BPW * 4)], q_f, qsem)
    tcp = pltpu.async_copy(t_hbm.at[pl.ds(0, BPW * 3)], t_f, tsem)
    qcp.wait()
    tcp.wait()

    lanes = lax.iota(jnp.int32, LANES)
    zeros = jnp.zeros((LANES,), jnp.float32)
    ones = jnp.ones((LANES,), jnp.float32)

    def body(g, carry):
        o = g * LANES
        x = q_f[pl.ds(o, LANES)]
        y = q_f[pl.ds(BPW + o, LANES)]
        z = q_f[pl.ds(2 * BPW + o, LANES)]
        w = q_f[pl.ds(3 * BPW + o, LANES)]
        tx = t_f[pl.ds(o, LANES)]
        ty = t_f[pl.ds(BPW + o, LANES)]
        tz = t_f[pl.ds(2 * BPW + o, LANES)]

        xx, yy, zz, ww = x * x, y * y, z * z, w * w
        s = xx + yy + zz + ww
        inv2 = 2.0 / jnp.maximum(s, 1e-24)
        xy, xz, yz = x * y, x * z, y * z
        xw, yw, zw = x * w, y * w, z * w

        entries = (
            1.0 - inv2 * (yy + zz),
            inv2 * (xy - zw),
            inv2 * (xz + yw),
            tx,
            inv2 * (xy + zw),
            1.0 - inv2 * (xx + zz),
            inv2 * (yz - xw),
            ty,
            inv2 * (xz - yw),
            inv2 * (yz + xw),
            1.0 - inv2 * (xx + yy),
            tz,
            zeros,
            zeros,
            zeros,
            ones,
        )
        ro = (o + lanes) * 16
        for e, val in enumerate(entries):
            plsc.store_scatter(out_f, [ro + e], val)
        return carry

    lax.fori_loop(0, 1, body, 0)
    pltpu.sync_copy(out_f, out_hbm.at[pl.ds(base * 16, BPW * 16)])


def kernel(indices, q, t):
    out = _pose_table_sc(
        indices.astype(jnp.int32),
        q.reshape(-1),
        t.reshape(-1),
    )
    return out.reshape(B, 4, 4)


# speedup vs baseline: 33.3635x; 33.3635x over previous
import functools

import jax
import jax.numpy as jnp
from jax import lax
from jax.experimental import pallas as pl
from jax.experimental.pallas import tpu as pltpu
from jax.experimental.pallas import tpu_sc as plsc

N_CORES = 2
N_SUBCORES = 16
LANES = 16
NW = N_CORES * N_SUBCORES

B = 16384
BPW = B // NW
GROUPS = BPW // LANES
BLK = 128
GPB = BLK // LANES

_mesh = plsc.VectorSubcoreMesh(core_axis_name="c", subcore_axis_name="s")


@functools.partial(
    pl.kernel,
    mesh=_mesh,
    out_type=jax.ShapeDtypeStruct((B * 16,), jnp.float32),
    compiler_params=pltpu.CompilerParams(needs_layout_passes=False),
    scratch_types=[
        pltpu.VMEM((BPW,), jnp.int32),
        pltpu.VMEM((BPW,), jnp.float32),
        pltpu.VMEM((BPW,), jnp.float32),
        pltpu.VMEM((BPW,), jnp.float32),
        pltpu.VMEM((BPW,), jnp.float32),
        pltpu.VMEM((BPW,), jnp.float32),
        pltpu.VMEM((BPW,), jnp.float32),
        pltpu.VMEM((BPW,), jnp.float32),
        pltpu.VMEM((BPW * 16,), jnp.float32),
        pltpu.SemaphoreType.DMA,
    ],
)
def _pose_table_sc(idx_hbm, qx_h, qy_h, qz_h, qw_h, tx_h, ty_h, tz_h, out_hbm,
                   idx_v, xg, yg, zg, wg, txg, tyg, tzg, out_v, sem):
    wid = lax.axis_index("s") * N_CORES + lax.axis_index("c")
    base = wid * BPW

    pltpu.sync_copy(idx_hbm.at[pl.ds(base, BPW)], idx_v)
    copies = [
        pltpu.async_copy(qx_h.at[idx_v], xg, sem),
        pltpu.async_copy(qy_h.at[idx_v], yg, sem),
        pltpu.async_copy(qz_h.at[idx_v], zg, sem),
        pltpu.async_copy(qw_h.at[idx_v], wg, sem),
        pltpu.async_copy(tx_h.at[idx_v], txg, sem),
        pltpu.async_copy(ty_h.at[idx_v], tyg, sem),
        pltpu.async_copy(tz_h.at[idx_v], tzg, sem),
    ]
    for cp in copies:
        cp.wait()

    zeros = jnp.zeros((LANES,), jnp.float32)
    ones = jnp.ones((LANES,), jnp.float32)

    def body(g, carry):
        o = g * LANES
        x = xg[pl.ds(o, LANES)]
        y = yg[pl.ds(o, LANES)]
        z = zg[pl.ds(o, LANES)]
        w = wg[pl.ds(o, LANES)]
        tx = txg[pl.ds(o, LANES)]
        ty = tyg[pl.ds(o, LANES)]
        tz = tzg[pl.ds(o, LANES)]

        xx, yy, zz, ww = x * x, y * y, z * z, w * w
        s = xx + yy + zz + ww
        inv2 = 2.0 / jnp.maximum(s, 1e-24)
        xy, xz, yz = x * y, x * z, y * z
        xw, yw, zw = x * w, y * w, z * w

        entries = (
            (1.0 - inv2 * (yy + zz), 0, 0),
            (inv2 * (xy - zw), 0, 1),
            (inv2 * (xz + yw), 0, 2),
            (tx, 0, 3),
            (inv2 * (xy + zw), 1, 0),
            (1.0 - inv2 * (xx + zz), 1, 1),
            (inv2 * (yz - xw), 1, 2),
            (ty, 1, 3),
            (inv2 * (xz - yw), 2, 0),
            (inv2 * (yz + xw), 2, 1),
            (1.0 - inv2 * (xx + yy), 2, 2),
            (tz, 2, 3),
            (zeros, 3, 0),
            (zeros, 3, 1),
            (zeros, 3, 2),
            (ones, 3, 3),
        )
        blk = g // GPB
        off = g % GPB
        for val, r, c in entries:
            pos = r * (4 * BPW) + blk * (4 * BLK) + c * BLK + off * LANES
            out_v[pl.ds(pos, LANES)] = val
        return carry

    lax.fori_loop(0, GROUPS, body, 0)

    for r in range(4):
        pltpu.sync_copy(
            out_v.at[pl.ds(r * (4 * BPW), 4 * BPW)],
            out_hbm.at[pl.ds(r * (4 * B) + base * 4, 4 * BPW)],
        )


def kernel(indices, q, t):
    out = _pose_table_sc(
        indices.astype(jnp.int32),
        q[:, 0], q[:, 1], q[:, 2], q[:, 3],
        t[:, 0], t[:, 1], t[:, 2],
    )
    return (
        out.reshape(4, B // BLK, 4, BLK)
        .transpose(1, 3, 0, 2)
        .reshape(B, 4, 4)
    )
